# Initial kernel scaffold; baseline (speedup 1.0000x reference)
import jax
import jax.numpy as jnp
from jax.experimental import pallas as pl


def kernel(x, pos, labels, params):
    raise NotImplementedError("write your pallas kernel here")


# speedup vs baseline: 1.0016x; 1.0016x over previous
import functools

import jax
import jax.numpy as jnp
import numpy as np
from jax.experimental import pallas as pl


def _ap(p, x):
    y = x @ p['W']
    if p['b'] is not None:
        y = y + p['b']
    return y


def _cdist(a, b):
    d2 = jnp.sum(a * a, -1)[:, :, None] + jnp.sum(b * b, -1)[:, None, :] - 2.0 * jnp.einsum('bnd,bmd->bnm', a, b)
    return jnp.sqrt(jnp.maximum(d2, 0.0))


def _knn_idx(cd, k):
    return jax.lax.top_k(-cd, k)[1]


def _gather(x, idx):
    return jax.vmap(lambda xb, ib: xb[ib])(x, idx)


def _edge_conv(p, x, idx):
    xj = _gather(x, idx)
    xi = x[:, :, None, :]
    e = jnp.concatenate([jnp.broadcast_to(xi, xj.shape), xj - xi], -1)
    h = jax.nn.relu(_ap(p['m1'], e))
    h = jax.nn.relu(_ap(p['m2'], h))
    return jnp.max(h, axis=2)


def _ln(p, x):
    m = jnp.mean(x, -1, keepdims=True)
    v = jnp.var(x, -1, keepdims=True)
    return (x - m) / jnp.sqrt(v + 1e-5) * p['g'] + p['b']


def _stn(p, x):
    h = jax.nn.relu(_ap(p['c1'], x))
    h = jax.nn.relu(_ap(p['c2'], h))
    h = jax.nn.relu(_ap(p['c3'], h))
    g = jnp.max(h, axis=1)
    g = jax.nn.relu(_ap(p['f1'], g))
    g = jax.nn.relu(_ap(p['f2'], g))
    t = _ap(p['f3'], g) + jnp.eye(24, dtype=jnp.float32).reshape(-1)
    t = t.reshape(-1, 24, 24)
    return jnp.einsum('bnc,bcd->bnd', x, t)


def _dilated_idx(cd, dilation_k, k):
    idx = jax.lax.top_k(-cd, dilation_k)[1]
    step = dilation_k // k
    return idx[:, :, ::step][:, :, :k]


def _head_kernel(xf_ref, wfi_ref,
                 w1a_ref, b1a_ref, w1b_ref, b1b_ref, w1r_ref, b1r_ref,
                 w2a_ref, b2a_ref, w2b_ref, b2b_ref, w2r_ref, b2r_ref,
                 wo_ref, bo_ref,
                 seg_ref, feat_ref):
    xf = xf_ref[...]
    xg = xf * jax.nn.sigmoid(jnp.dot(xf, wfi_ref[...], preferred_element_type=jnp.float32))
    h = jax.nn.relu(jnp.dot(xg, w1a_ref[...], preferred_element_type=jnp.float32) + b1a_ref[...])
    r1 = jax.nn.relu(jnp.dot(h, w1b_ref[...], preferred_element_type=jnp.float32) + b1b_ref[...])
    r1 = r1 + jnp.dot(xg, w1r_ref[...], preferred_element_type=jnp.float32) + b1r_ref[...]
    h2 = jax.nn.relu(jnp.dot(r1, w2a_ref[...], preferred_element_type=jnp.float32) + b2a_ref[...])
    feat = jax.nn.relu(jnp.dot(h2, w2b_ref[...], preferred_element_type=jnp.float32) + b2b_ref[...])
    feat = feat + jnp.dot(r1, w2r_ref[...], preferred_element_type=jnp.float32) + b2r_ref[...]
    feat_ref[...] = feat
    seg_ref[...] = jnp.dot(feat, wo_ref[...], preferred_element_type=jnp.float32) + bo_ref[...]


def _run_head(x_fused, params):
    B, N = x_fused.shape[0], x_fused.shape[1]
    M = B * N
    xf = x_fused.reshape(M, 256)
    p = params
    wo = jnp.zeros((256, 128), jnp.float32).at[:, :17].set(p['out']['W'])
    bo = jnp.zeros((1, 128), jnp.float32).at[0, :17].set(p['out']['b'])
    BM = 1024
    grid = (M // BM,)
    row_spec = lambda c: pl.BlockSpec((BM, c), lambda i: (i, 0))
    args = [xf,
            p['fi']['W'],
            p['rb1a']['W'], p['rb1a']['b'].reshape(1, -1),
            p['rb1b']['W'], p['rb1b']['b'].reshape(1, -1),
            p['rb1r']['W'], p['rb1r']['b'].reshape(1, -1),
            p['rb2a']['W'], p['rb2a']['b'].reshape(1, -1),
            p['rb2b']['W'], p['rb2b']['b'].reshape(1, -1),
            p['rb2r']['W'], p['rb2r']['b'].reshape(1, -1),
            wo, bo]
    full = lambda a: pl.BlockSpec(a.shape, lambda i: tuple(0 for _ in a.shape))
    in_specs = [row_spec(256)] + [full(a) for a in args[1:]]
    seg, feat = pl.pallas_call(
        _head_kernel,
        grid=grid,
        in_specs=in_specs,
        out_specs=[row_spec(128), row_spec(256)],
        out_shape=[jax.ShapeDtypeStruct((M, 128), jnp.float32),
                   jax.ShapeDtypeStruct((M, 256), jnp.float32)],
    )(*args)
    return seg[:, :17].reshape(B, N, 17), feat.reshape(B, N, 256)


def kernel(x, pos, labels, params):
    cd = _cdist(pos, pos)
    x = _stn(params['stn'], x)
    x1 = _edge_conv(params['e1'], x, _knn_idx(cd, 32))
    x2 = _edge_conv(params['e2'], x1, _knn_idx(_cdist(x1, x1), 32))
    x3 = _edge_conv(params['e3'], x2, _knn_idx(_cdist(x2, x2), 32))
    x_local = jnp.concatenate([x1, x2, x3], -1)
    x_mid = jax.nn.relu(_ap(params['local_hidden'], x_local))
    xd1 = _edge_conv(params['d1'], x_mid, _dilated_idx(cd, 200, 32))
    xd2 = _edge_conv(params['d2'], xd1, _dilated_idx(cd, 900, 32))
    xd3 = _edge_conv(params['d3'], xd2, _dilated_idx(cd, 1800, 32))
    x_global = jnp.concatenate([xd1, xd2, xd3], -1)
    x_temp = jnp.concatenate([x_mid, xd1, xd2, xd3], -1)
    logits_temp = _ap(params['temp2'], jax.nn.relu(_ln(params['temp_ln'], _ap(params['temp1'], x_temp))))
    f0 = _ap(params['proj0'], x_local)
    f1 = _ap(params['proj1'], x_mid)
    f2 = _ap(params['proj2'], x_global)
    fs = jnp.stack([f0, f1, f2], axis=2)
    tl = jnp.argmax(logits_temp, -1)
    nidx = _knn_idx(cd, 9)[:, :, 1:]
    nl = jax.vmap(lambda lb, ib: lb[ib])(tl, nidx)
    diff = jnp.mean((nl != tl[:, :, None]).astype(jnp.float32), -1)
    probs = jax.nn.softmax(logits_temp, -1)
    conf = jnp.max(probs, -1)
    ent = -jnp.sum(probs * jnp.log(probs + 1e-8), -1) / np.log(probs.shape[-1])
    binfo = jnp.stack([diff, conf, ent], -1)
    benc = _ap(params['be2'], jax.nn.relu(_ap(params['be1'], binfo)))
    gfeat = jnp.mean(fs, axis=2)
    aw = jax.nn.softmax(_ap(params['at2'], jax.nn.relu(_ap(params['at1'], jnp.concatenate([gfeat, benc], -1)))), -1)
    fused = jnp.sum(fs * aw[:, :, :, None], axis=2)
    x_fused = _ap(params['op2'], jax.nn.relu(_ap(params['op1'], fused))) + gfeat
    seg_pred, features = _run_head(x_fused, params)
    return (seg_pred, features, x_fused)
